# baseline (speedup 1.0000x reference)
import jax
import jax.numpy as jnp
from jax.experimental import pallas as pl
from jax.experimental.pallas import tpu as pltpu

TM = 256


def _gating_kernel(x_ref, wg_ref, g_ref):
    x = x_ref[...]
    wg = wg_ref[...]
    s = jax.lax.dot_general(x, wg, (((1,), (1,)), ((), ())),
                            preferred_element_type=jnp.float32)
    E = s.shape[-1]
    ids = jax.lax.broadcasted_iota(jnp.int32, s.shape, 1)
    m1 = jnp.max(s, axis=-1, keepdims=True)
    i1 = jnp.min(jnp.where(s == m1, ids, E), axis=-1, keepdims=True)
    s_masked = jnp.where(ids == i1, -jnp.inf, s)
    m2 = jnp.max(s_masked, axis=-1, keepdims=True)
    i2 = jnp.min(jnp.where(s_masked == m2, ids, E), axis=-1, keepdims=True)
    p1 = jax.nn.sigmoid(m1 - m2)
    p2 = 1.0 - p1
    g_ref[...] = jnp.where(ids == i1, p1, 0.0) + jnp.where(ids == i2, p2, 0.0)


def _moe_kernel(x_ref, g_ref, w1_ref, w2_ref, w3_ref, o_ref):
    e = pl.program_id(1)
    xb = x_ref[...]
    h1 = jnp.dot(xb, w1_ref[0], preferred_element_type=jnp.float32)
    h2 = jnp.dot(xb, w2_ref[0], preferred_element_type=jnp.float32)
    h = (h1 * jax.nn.sigmoid(h1)) * h2
    y = jnp.dot(h, w3_ref[0], preferred_element_type=jnp.float32)
    g = g_ref[...]
    lane = jax.lax.broadcasted_iota(jnp.int32, g.shape, 1)
    ge = jnp.sum(jnp.where(lane == e, g, 0.0), axis=-1, keepdims=True)
    acc = ge * y

    @pl.when(e == 0)
    def _():
        o_ref[...] = acc

    @pl.when(e != 0)
    def _():
        o_ref[...] += acc


def kernel(x, Wg, W1, W2, W3):
    B, T, D = x.shape
    E, _, F = W1.shape
    x2 = x.reshape(T, D)

    g = pl.pallas_call(
        _gating_kernel,
        out_shape=jax.ShapeDtypeStruct((T, E), jnp.float32),
    )(x2, Wg)

    nt = T // TM
    out = pl.pallas_call(
        _moe_kernel,
        grid=(nt, E),
        in_specs=[
            pl.BlockSpec((TM, D), lambda i, e: (i, 0)),
            pl.BlockSpec((TM, E), lambda i, e: (i, 0)),
            pl.BlockSpec((1, D, F), lambda i, e: (e, 0, 0)),
            pl.BlockSpec((1, D, F), lambda i, e: (e, 0, 0)),
            pl.BlockSpec((1, F, D), lambda i, e: (e, 0, 0)),
        ],
        out_specs=pl.BlockSpec((TM, D), lambda i, e: (i, 0)),
        out_shape=jax.ShapeDtypeStruct((T, D), jnp.float32),
        compiler_params=pltpu.CompilerParams(
            dimension_semantics=("arbitrary", "arbitrary"),
        ),
    )(x2, g, W1, W2, W3)

    return out.reshape(B, T, D)


# speedup vs baseline: 1.3757x; 1.3757x over previous
import jax
import jax.numpy as jnp
from jax.experimental import pallas as pl
from jax.experimental.pallas import tpu as pltpu

TM = 256


def _gating_kernel(x_ref, wg_ref, g_ref):
    x = x_ref[...]
    wg = wg_ref[...]
    s = jax.lax.dot_general(x, wg, (((1,), (1,)), ((), ())),
                            preferred_element_type=jnp.float32)
    E = s.shape[-1]
    ids = jax.lax.broadcasted_iota(jnp.int32, s.shape, 1)
    m1 = jnp.max(s, axis=-1, keepdims=True)
    i1 = jnp.min(jnp.where(s == m1, ids, E), axis=-1, keepdims=True)
    s_masked = jnp.where(ids == i1, -jnp.inf, s)
    m2 = jnp.max(s_masked, axis=-1, keepdims=True)
    i2 = jnp.min(jnp.where(s_masked == m2, ids, E), axis=-1, keepdims=True)
    p1 = jax.nn.sigmoid(m1 - m2)
    p2 = 1.0 - p1
    g_ref[...] = jnp.where(ids == i1, p1, 0.0) + jnp.where(ids == i2, p2, 0.0)


def _moe_kernel(x_ref, g_ref, w1_ref, w2_ref, w3_ref, o_ref, acc_ref):
    e = pl.program_id(0)
    i = pl.program_id(1)
    ne = pl.num_programs(0)
    xb = x_ref[...].astype(jnp.bfloat16)
    h1 = jnp.dot(xb, w1_ref[0], preferred_element_type=jnp.float32)
    h2 = jnp.dot(xb, w2_ref[0], preferred_element_type=jnp.float32)
    h = ((h1 * jax.nn.sigmoid(h1)) * h2).astype(jnp.bfloat16)
    y = jnp.dot(h, w3_ref[0], preferred_element_type=jnp.float32)
    g = g_ref[...]
    lane = jax.lax.broadcasted_iota(jnp.int32, g.shape, 1)
    ge = jnp.sum(jnp.where(lane == e, g, 0.0), axis=-1, keepdims=True)
    acc = ge * y
    rows = pl.ds(i * x_ref.shape[0], x_ref.shape[0])

    @pl.when(e == 0)
    def _():
        acc_ref[rows, :] = acc

    @pl.when(e != 0)
    def _():
        acc_ref[rows, :] += acc

    @pl.when(e == ne - 1)
    def _():
        o_ref[...] = acc_ref[rows, :]


def kernel(x, Wg, W1, W2, W3):
    B, T, D = x.shape
    E, _, F = W1.shape
    x2 = x.reshape(T, D)

    g = pl.pallas_call(
        _gating_kernel,
        out_shape=jax.ShapeDtypeStruct((T, E), jnp.float32),
    )(x2, Wg)

    nt = T // TM
    out = pl.pallas_call(
        _moe_kernel,
        grid=(E, nt),
        in_specs=[
            pl.BlockSpec((TM, D), lambda e, i: (i, 0)),
            pl.BlockSpec((TM, E), lambda e, i: (i, 0)),
            pl.BlockSpec((1, D, F), lambda e, i: (e, 0, 0)),
            pl.BlockSpec((1, D, F), lambda e, i: (e, 0, 0)),
            pl.BlockSpec((1, F, D), lambda e, i: (e, 0, 0)),
        ],
        out_specs=pl.BlockSpec((TM, D), lambda e, i: (i, 0)),
        out_shape=jax.ShapeDtypeStruct((T, D), jnp.float32),
        scratch_shapes=[pltpu.VMEM((T, D), jnp.float32)],
        compiler_params=pltpu.CompilerParams(
            dimension_semantics=("arbitrary", "arbitrary"),
        ),
    )(x2, g, W1.astype(jnp.bfloat16), W2.astype(jnp.bfloat16),
      W3.astype(jnp.bfloat16))

    return out.reshape(B, T, D)
